# baseline (speedup 1.0000x reference)
import functools

import jax
import jax.numpy as jnp
from jax import lax
from jax.experimental import pallas as pl
from jax.experimental.pallas import tpu as pltpu
from jax.experimental.pallas import tpu_sc as plsc

_NEG_INF = float(-1e37)
_IDX_CHUNK = 128


def _chunks(n):
    out, off = [], 0
    while off < n:
        sz = min(_IDX_CHUNK, n - off)
        out.append((off, sz))
        off += sz
    return out


@functools.partial(jax.jit, static_argnames=("naug",))
def _sc_gather(weight, bias16, aug_idx, label_idx, *, naug):
    v16, lanes = bias16.shape
    d = weight.shape[1]
    b = label_idx.shape[0]
    info = plsc.get_sparse_core_info()
    nc, nsc = info.num_cores, info.num_subcores
    nw = nc * nsc
    n1 = naug // nw
    n2 = b // nw
    ch1, ch2 = _chunks(n1), _chunks(n2)

    mesh = plsc.VectorSubcoreMesh(core_axis_name="c", subcore_axis_name="s")

    @functools.partial(
        pl.kernel,
        out_type=(
            jax.ShapeDtypeStruct((naug, d), jnp.float32),
            jax.ShapeDtypeStruct((naug,), jnp.float32),
            jax.ShapeDtypeStruct((b, d), jnp.float32),
            jax.ShapeDtypeStruct((b,), jnp.float32),
        ),
        mesh=mesh,
        compiler_params=pltpu.CompilerParams(
            use_tc_tiling_on_sc=False, needs_layout_passes=False),
        scratch_types=(
            pltpu.VMEM((n1,), jnp.int32),
            pltpu.VMEM((n1,), jnp.int32),
            pltpu.VMEM((n1,), jnp.int32),
            pltpu.VMEM((n1, d), jnp.float32),
            pltpu.VMEM((n1, 16), jnp.float32),
            pltpu.VMEM((n1,), jnp.float32),
            pltpu.VMEM((n2,), jnp.int32),
            pltpu.VMEM((n2,), jnp.int32),
            pltpu.VMEM((n2,), jnp.int32),
            pltpu.VMEM((n2, d), jnp.float32),
            pltpu.VMEM((n2, 16), jnp.float32),
            pltpu.VMEM((n2,), jnp.float32),
            pltpu.SemaphoreType.DMA,
        ),
    )
    def gather(w_hbm, b16_hbm, aidx_hbm, lidx_hbm,
               wa_out, ba_out, tw_out, tb_out,
               idx1, idx1h, idx1l, rows1, b16_1, bsel1,
               idx2, idx2h, idx2l, rows2, b16_2, bsel2, sem):
        iota16 = lax.iota(jnp.int32, 16)
        wid = lax.axis_index("s") * nc + lax.axis_index("c")
        base1 = wid * n1
        base2 = wid * n2
        pltpu.sync_copy(aidx_hbm.at[pl.ds(base1, n1)], idx1)
        pltpu.sync_copy(lidx_hbm.at[pl.ds(base2, n2)], idx2)
        for idx, idxh, idxl, n in ((idx1, idx1h, idx1l, n1),
                                   (idx2, idx2h, idx2l, n2)):
            for g in range(n // 16):
                sl = pl.ds(16 * g, 16)
                val = idx[sl]
                idxh[sl] = lax.shift_right_logical(val, 4)
                idxl[sl] = lax.bitwise_and(val, 15)
        copies = []
        for off, sz in ch1:
            copies.append(pltpu.async_copy(
                w_hbm.at[idx1.at[pl.ds(off, sz)]], rows1.at[pl.ds(off, sz)], sem))
            copies.append(pltpu.async_copy(
                b16_hbm.at[idx1h.at[pl.ds(off, sz)]], b16_1.at[pl.ds(off, sz)], sem))
        for off, sz in ch2:
            copies.append(pltpu.async_copy(
                w_hbm.at[idx2.at[pl.ds(off, sz)]], rows2.at[pl.ds(off, sz)], sem))
            copies.append(pltpu.async_copy(
                b16_hbm.at[idx2h.at[pl.ds(off, sz)]], b16_2.at[pl.ds(off, sz)], sem))
        for c in copies:
            c.wait()
        for idxl, b16v, bsel, n in ((idx1l, b16_1, bsel1, n1),
                                    (idx2l, b16_2, bsel2, n2)):
            for g in range(n // 16):
                sl = pl.ds(16 * g, 16)
                rows = 16 * g + iota16
                bsel[sl] = plsc.load_gather(b16v, [rows, idxl[sl]])
        pltpu.sync_copy(rows1, wa_out.at[pl.ds(base1, n1)])
        pltpu.sync_copy(bsel1, ba_out.at[pl.ds(base1, n1)])
        pltpu.sync_copy(rows2, tw_out.at[pl.ds(base2, n2)])
        pltpu.sync_copy(bsel2, tb_out.at[pl.ds(base2, n2)])

    return gather(weight, bias16, aug_idx, label_idx)


def _tc_body(x_ref, w_ref, ba_ref, sf_ref, ids_ref, lab_ref,
             tw_ref, tb_ref, tf_ref, out_ref, *, nout):
    x = x_ref[...]
    acc = lax.dot_general(x, w_ref[...], (((1,), (1,)), ((), ())),
                          preferred_element_type=jnp.float32)
    val = acc + (ba_ref[...] - jnp.log(sf_ref[...]))
    val = jnp.where(lab_ref[...] == ids_ref[...], _NEG_INF, val)
    t = (jnp.sum(x * tw_ref[...], axis=1, keepdims=True)
         + tb_ref[...] - jnp.log(tf_ref[...]))
    col0 = lax.broadcasted_iota(jnp.int32, val.shape, 1) == 0
    val = jnp.where(col0, t, val)
    out_ref[...] = val[:, :nout]


def kernel(inputs, labels, weight, bias, sample_ids, true_freq, sample_freq):
    b, d = inputs.shape
    ns = sample_ids.shape[0]
    v = weight.shape[0]
    nout = ns + 1

    labels_i = labels.astype(jnp.int32)
    sids_i = sample_ids.astype(jnp.int32)

    info = plsc.get_sparse_core_info()
    nw = info.num_cores * info.num_subcores
    step = max(16 * nw, 128)
    naug = ((nout + step - 1) // step) * step

    pad = naug - 1 - ns
    one_i = jnp.zeros((1,), jnp.int32)
    aug_idx = jnp.concatenate([one_i, sids_i, jnp.zeros((pad,), jnp.int32)])
    aug_ids = jnp.concatenate(
        [jnp.full((1,), -1, jnp.int32), sids_i, jnp.full((pad,), -1, jnp.int32)]
    ).reshape(1, naug)
    aug_sf = jnp.concatenate(
        [jnp.ones((1,), jnp.float32), sample_freq, jnp.ones((pad,), jnp.float32)]
    ).reshape(1, naug)

    w_aug, b_aug, tw, tb = _sc_gather(
        weight, bias.reshape(v // 16, 16), aug_idx, labels_i, naug=naug)
    b_aug = b_aug.reshape(1, naug)
    tb = tb.reshape(b, 1)

    br = 256
    grid = (b // br,)
    out = pl.pallas_call(
        functools.partial(_tc_body, nout=nout),
        grid=grid,
        in_specs=[
            pl.BlockSpec((br, d), lambda i: (i, 0)),
            pl.BlockSpec((naug, d), lambda i: (0, 0)),
            pl.BlockSpec((1, naug), lambda i: (0, 0)),
            pl.BlockSpec((1, naug), lambda i: (0, 0)),
            pl.BlockSpec((1, naug), lambda i: (0, 0)),
            pl.BlockSpec((br, 1), lambda i: (i, 0)),
            pl.BlockSpec((br, d), lambda i: (i, 0)),
            pl.BlockSpec((br, 1), lambda i: (i, 0)),
            pl.BlockSpec((br, 1), lambda i: (i, 0)),
        ],
        out_specs=pl.BlockSpec((br, nout), lambda i: (i, 0)),
        out_shape=jax.ShapeDtypeStruct((b, nout), jnp.float32),
    )(inputs, w_aug, b_aug, aug_sf, aug_ids, labels_i.reshape(b, 1),
      tw, tb, true_freq.reshape(b, 1))
    return out
